# baseline (speedup 1.0000x reference)
import functools
import math

import jax
import jax.numpy as jnp
from jax.experimental import pallas as pl
from jax.experimental.pallas import tpu as pltpu

N_BPS = 4096
HIDDEN = 256
ATTN = 64
VAL = 128
K_NN = 16
N_LAYERS = 2
N_CLASSES = 5


def _gelu(x):
    return 0.5 * x * (1.0 + jax.lax.erf(x * (1.0 / math.sqrt(2.0))))


def _dot(a, b):
    return jnp.dot(a, b, preferred_element_type=jnp.float32)


def _ln(x, g, b):
    m = jnp.mean(x, axis=-1, keepdims=True)
    v = jnp.mean((x - m) ** 2, axis=-1, keepdims=True)
    return (x - m) * jax.lax.rsqrt(v + 1e-5) * g + b


def _fwd_kernel(p_ref, *refs):
    out_ref = refs[-1]
    w = [r[...] for r in refs[:-1]]
    it = iter(w)
    nxt = lambda: next(it)

    A8, b0, W1, b1 = nxt(), nxt(), nxt(), nxt()
    p = p_ref[0]
    x = jnp.maximum(_dot(p, A8) + b0, 0.0)
    x = _dot(x, W1) + b1

    for _ in range(N_LAYERS):
        g1, bn1, Wq, bq, Wk, bkp, Wv, bvp = (nxt() for _ in range(8))
        Ws0, bs0 = nxt(), nxt()
        ws1 = [nxt() for _ in range(K_NN // 4)]
        Wout, bout, g2, bn2, Wf1, bf1, Wf2, bf2 = (nxt() for _ in range(8))

        h = _ln(x, g1, bn1)
        q = _dot(h, Wq) + bq
        hs = h[1:K_NN + 1, :]
        Kp = _dot(hs, Wk) + bkp
        Kp2 = [jnp.concatenate([Kp[4 * pj + i:4 * pj + i + 1, :]
                                for i in range(4)], axis=1)
               for pj in range(K_NN // 4)]
        Vp = _dot(hs, Wv) + bvp

        q2 = jnp.concatenate([q, q, q, q], axis=1)
        s = None
        for pj in range(K_NN // 4):
            t2 = jnp.tanh(q2 - Kp2[pj])
            u2 = jnp.maximum(_dot(t2, Ws0) + bs0, 0.0)
            sj = _dot(u2, ws1[pj])
            s = sj if s is None else s + sj
        s = s * (1.0 / math.sqrt(ATTN))
        s = s - jnp.max(s, axis=1, keepdims=True)
        e = jnp.exp(s)
        attn = e / jnp.sum(e, axis=1, keepdims=True)
        ctx = _dot(attn, Vp)
        x = x + _dot(ctx, Wout) + bout

        h2 = _ln(x, g2, bn2)
        f = _dot(h2, Wf1) + bf1
        f = _gelu(f)
        x = x + _dot(f, Wf2) + bf2

    Wg1, bg1, Wg2, bg2, hg, hb, Wh0, bh0, Wh1, bh1 = (nxt() for _ in range(10))
    g = jnp.mean(x, axis=0, keepdims=True)
    g = _dot(g, Wg1) + bg1
    g = _gelu(g)
    g = _dot(g, Wg2) + bg2
    fused = jnp.concatenate([x, jnp.broadcast_to(g, (N_BPS, HIDDEN))], axis=1)
    hh = _ln(fused, hg, hb)
    hh = jnp.maximum(_dot(hh, Wh0) + bh0, 0.0)
    out_ref[0] = _dot(hh, Wh1) + bh1


def _row(v):
    return v.reshape(1, -1)


def kernel(bps_dists, bps_nn_points, params):
    Bsz = bps_dists.shape[0]

    W0 = params["input_proj"][0]["w"]
    A = jnp.concatenate([W0[3:6] - W0[6:9], W0[9:10]], axis=0)
    A8 = jnp.pad(A, ((0, 4), (0, 0)))
    P4 = jnp.concatenate([bps_nn_points, bps_dists[..., None]], axis=-1)
    P8 = jnp.pad(P4, ((0, 0), (0, 0), (0, 4)))

    ws = [A8, _row(params["input_proj"][0]["b"]),
          params["input_proj"][1]["w"], _row(params["input_proj"][1]["b"])]

    for blk in params["blocks"]:
        c_a = (jnp.maximum(blk["pos_attn"][0]["b"], 0.0) @ blk["pos_attn"][1]["w"]
               + blk["pos_attn"][1]["b"])
        c_v = (jnp.maximum(blk["pos_value"][0]["b"], 0.0) @ blk["pos_value"][1]["w"]
               + blk["pos_value"][1]["b"])
        ws += [
            _row(blk["ln1"]["g"]), _row(blk["ln1"]["b"]),
            blk["q"]["w"], _row(blk["q"]["b"]),
            blk["k"]["w"], _row(blk["k"]["b"] - c_a),
            blk["v"]["w"], _row(blk["v"]["b"] + c_v),
            jax.scipy.linalg.block_diag(*([blk["score"][0]["w"]] * 4)),
            _row(jnp.tile(blk["score"][0]["b"], 4)),
            *[jnp.concatenate(
                [jnp.outer(blk["score"][1]["w"][:, 0],
                           jnp.zeros((K_NN,)).at[4 * pj + i].set(1.0))
                 for i in range(4)],
                axis=0)
              for pj in range(K_NN // 4)],
            blk["out"]["w"], _row(blk["out"]["b"]),
            _row(blk["ln2"]["g"]), _row(blk["ln2"]["b"]),
            blk["ffn"][0]["w"], _row(blk["ffn"][0]["b"]),
            blk["ffn"][1]["w"], _row(blk["ffn"][1]["b"]),
        ]

    Wh1 = jnp.pad(params["head"][1]["w"], ((0, 0), (0, 8 - N_CLASSES)))
    bh1 = jnp.pad(_row(params["head"][1]["b"]), ((0, 0), (0, 8 - N_CLASSES)))
    ws += [
        params["global_proj"][0]["w"], _row(params["global_proj"][0]["b"]),
        params["global_proj"][1]["w"], _row(params["global_proj"][1]["b"]),
        _row(params["head_ln"]["g"]), _row(params["head_ln"]["b"]),
        params["head"][0]["w"], _row(params["head"][0]["b"]),
        Wh1, bh1,
    ]

    in_specs = [pl.BlockSpec((1, N_BPS, 8), lambda b: (b, 0, 0))]
    for a in ws:
        nd = a.ndim
        in_specs.append(pl.BlockSpec(a.shape, lambda b, _nd=nd: (0,) * _nd))

    out = pl.pallas_call(
        _fwd_kernel,
        grid=(Bsz,),
        in_specs=in_specs,
        out_specs=pl.BlockSpec((1, N_BPS, 8), lambda b: (b, 0, 0)),
        out_shape=jax.ShapeDtypeStruct((Bsz, N_BPS, 8), jnp.float32),
        compiler_params=pltpu.CompilerParams(
            dimension_semantics=("parallel",)),
    )(P8, *ws)
    return out[..., :N_CLASSES]


# speedup vs baseline: 1.0790x; 1.0790x over previous
import functools
import math

import jax
import jax.numpy as jnp
from jax.experimental import pallas as pl
from jax.experimental.pallas import tpu as pltpu

N_BPS = 4096
HIDDEN = 256
ATTN = 64
VAL = 128
K_NN = 16
N_LAYERS = 2
N_CLASSES = 5


def _gelu(x):
    return 0.5 * x * (1.0 + jax.lax.erf(x * (1.0 / math.sqrt(2.0))))


def _dot(a, b):
    return jnp.dot(a, b, preferred_element_type=jnp.float32)


def _ln(x, g, b):
    m = jnp.mean(x, axis=-1, keepdims=True)
    v = jnp.mean((x - m) ** 2, axis=-1, keepdims=True)
    return (x - m) * jax.lax.rsqrt(v + 1e-5) * g + b


def _fwd_kernel(p_ref, *refs):
    out_ref = refs[-1]
    w = [r[...] for r in refs[:-1]]
    it = iter(w)
    nxt = lambda: next(it)

    A8, b0, W1, b1 = nxt(), nxt(), nxt(), nxt()
    p = p_ref[0]
    x = jnp.maximum(_dot(p, A8) + b0, 0.0)
    x = _dot(x, W1) + b1

    for _ in range(N_LAYERS):
        g1, bn1, Wq, bq, Wk, bkp, Wv, bvp = (nxt() for _ in range(8))
        Ws0, bs0 = nxt(), nxt()
        ws1 = [nxt() for _ in range(K_NN // 2)]
        Wout, bout, g2, bn2, Wf1, bf1, Wf2, bf2 = (nxt() for _ in range(8))

        h = _ln(x, g1, bn1)
        q = _dot(h, Wq) + bq
        hs = h[1:K_NN + 1, :]
        Kp = _dot(hs, Wk) + bkp
        Kp2 = [jnp.concatenate([Kp[2 * pj:2 * pj + 1, :],
                                Kp[2 * pj + 1:2 * pj + 2, :]], axis=1)
               for pj in range(K_NN // 2)]
        Vp = _dot(hs, Wv) + bvp

        q2 = jnp.concatenate([q, q], axis=1)
        s = None
        for pj in range(K_NN // 2):
            t2 = jnp.tanh(q2 - Kp2[pj])
            u2 = jnp.maximum(_dot(t2, Ws0) + bs0, 0.0)
            sj = _dot(u2, ws1[pj])
            s = sj if s is None else s + sj
        s = s * (1.0 / math.sqrt(ATTN))
        e = jnp.exp(s)
        attn = e / jnp.sum(e, axis=1, keepdims=True)
        ctx = _dot(attn, Vp)
        x = x + _dot(ctx, Wout) + bout

        h2 = _ln(x, g2, bn2)
        f = _dot(h2, Wf1) + bf1
        f = _gelu(f)
        x = x + _dot(f, Wf2) + bf2

    Wg1, bg1, Wg2, bg2, hg, hb, Wh0, bh0, Wh1, bh1 = (nxt() for _ in range(10))
    g = jnp.mean(x, axis=0, keepdims=True)
    g = _dot(g, Wg1) + bg1
    g = _gelu(g)
    g = _dot(g, Wg2) + bg2
    fused = jnp.concatenate([x, jnp.broadcast_to(g, (N_BPS, HIDDEN))], axis=1)
    hh = _ln(fused, hg, hb)
    hh = jnp.maximum(_dot(hh, Wh0) + bh0, 0.0)
    out_ref[0] = _dot(hh, Wh1) + bh1


def _row(v):
    return v.reshape(1, -1)


def kernel(bps_dists, bps_nn_points, params):
    Bsz = bps_dists.shape[0]

    W0 = params["input_proj"][0]["w"]
    A = jnp.concatenate([W0[3:6] - W0[6:9], W0[9:10]], axis=0)
    A8 = jnp.pad(A, ((0, 4), (0, 0)))
    P4 = jnp.concatenate([bps_nn_points, bps_dists[..., None]], axis=-1)
    P8 = jnp.pad(P4, ((0, 0), (0, 0), (0, 4)))

    ws = [A8, _row(params["input_proj"][0]["b"]),
          params["input_proj"][1]["w"], _row(params["input_proj"][1]["b"])]

    for blk in params["blocks"]:
        c_a = (jnp.maximum(blk["pos_attn"][0]["b"], 0.0) @ blk["pos_attn"][1]["w"]
               + blk["pos_attn"][1]["b"])
        c_v = (jnp.maximum(blk["pos_value"][0]["b"], 0.0) @ blk["pos_value"][1]["w"]
               + blk["pos_value"][1]["b"])
        ws += [
            _row(blk["ln1"]["g"]), _row(blk["ln1"]["b"]),
            blk["q"]["w"], _row(blk["q"]["b"]),
            blk["k"]["w"], _row(blk["k"]["b"] - c_a),
            blk["v"]["w"], _row(blk["v"]["b"] + c_v),
            jnp.block([[blk["score"][0]["w"], jnp.zeros((ATTN, ATTN))],
                       [jnp.zeros((ATTN, ATTN)), blk["score"][0]["w"]]]),
            _row(jnp.tile(blk["score"][0]["b"], 2)),
            *[jnp.concatenate(
                [jnp.outer(blk["score"][1]["w"][:, 0],
                           jnp.zeros((K_NN,)).at[2 * pj].set(1.0)),
                 jnp.outer(blk["score"][1]["w"][:, 0],
                           jnp.zeros((K_NN,)).at[2 * pj + 1].set(1.0))],
                axis=0)
              for pj in range(K_NN // 2)],
            blk["out"]["w"], _row(blk["out"]["b"]),
            _row(blk["ln2"]["g"]), _row(blk["ln2"]["b"]),
            blk["ffn"][0]["w"], _row(blk["ffn"][0]["b"]),
            blk["ffn"][1]["w"], _row(blk["ffn"][1]["b"]),
        ]

    Wh1 = jnp.pad(params["head"][1]["w"], ((0, 0), (0, 8 - N_CLASSES)))
    bh1 = jnp.pad(_row(params["head"][1]["b"]), ((0, 0), (0, 8 - N_CLASSES)))
    ws += [
        params["global_proj"][0]["w"], _row(params["global_proj"][0]["b"]),
        params["global_proj"][1]["w"], _row(params["global_proj"][1]["b"]),
        _row(params["head_ln"]["g"]), _row(params["head_ln"]["b"]),
        params["head"][0]["w"], _row(params["head"][0]["b"]),
        Wh1, bh1,
    ]

    in_specs = [pl.BlockSpec((1, N_BPS, 8), lambda b: (b, 0, 0))]
    for a in ws:
        nd = a.ndim
        in_specs.append(pl.BlockSpec(a.shape, lambda b, _nd=nd: (0,) * _nd))

    out = pl.pallas_call(
        _fwd_kernel,
        grid=(Bsz,),
        in_specs=in_specs,
        out_specs=pl.BlockSpec((1, N_BPS, 8), lambda b: (b, 0, 0)),
        out_shape=jax.ShapeDtypeStruct((Bsz, N_BPS, 8), jnp.float32),
        compiler_params=pltpu.CompilerParams(
            dimension_semantics=("parallel",)),
    )(P8, *ws)
    return out[..., :N_CLASSES]
